# Initial kernel scaffold; baseline (speedup 1.0000x reference)
import jax
import jax.numpy as jnp
from jax.experimental import pallas as pl


def kernel(x, x_v, node_attr, edge_index, W_node, R1_W1, R1_b1, R1_W2, r1_read, U1, R2_W1, R2_b1, R2_W2, r2_read):
    raise NotImplementedError("write your pallas kernel here")


# speedup vs baseline: 1.1213x; 1.1213x over previous
import functools

import jax
import jax.numpy as jnp
from jax.experimental import pallas as pl
from jax.experimental.pallas import tpu as pltpu

RC = 5.0
_BLK = 1280


def _silu(h):
    return h / (1.0 + jnp.exp(-h))


def _edge_stage_body(x_ref, xv_ref, w1a_ref, b1a_ref, w2a_ref,
                     w1b_ref, b1b_ref, w2b_ref, y_ref, r1_ref, r2_ref):
    r = x_ref[...]
    v = xv_ref[...]
    B = r.shape[0]
    rs = jnp.maximum(r, 1e-2)
    pref = jnp.sqrt(2.0 / RC) / rs
    n = (jnp.arange(8, dtype=jnp.int32) + 1).astype(jnp.float32)
    rb = jnp.sin(rs[..., None] * (n * (jnp.pi / RC))[None, None, :]) * pref[..., None]

    nv = v / (jnp.sqrt(jnp.sum(v * v, axis=-1, keepdims=True)) + 1e-9)
    xh = nv[..., 0]
    yh = nv[..., 1]
    zh = nv[..., 2]
    parts = [jnp.ones_like(xh), xh, yh, zh,
             xh * yh, yh * zh, 3.0 * zh * zh - 1.0, xh * zh, xh * xh - yh * yh]
    y_ref[...] = jnp.stack(parts + [jnp.zeros_like(xh)] * 7, axis=-1)

    w1a = w1a_ref[...]
    w2a = w2a_ref[...]
    w1b = w1b_ref[...]
    w2b = w2b_ref[...]
    b1a = b1a_ref[...]
    b1b = b1b_ref[...]
    for b in range(B):
        rb_b = rb[b]
        ha = _silu(jnp.dot(rb_b, w1a, preferred_element_type=jnp.float32) + b1a)
        r1_ref[b] = jnp.dot(ha, w2a, preferred_element_type=jnp.float32)
        hb = _silu(jnp.dot(rb_b, w1b, preferred_element_type=jnp.float32) + b1b)
        r2_ref[b] = jnp.dot(hb, w2b, preferred_element_type=jnp.float32)


def _edge_stage(x, x_v, R1_W1, R1_b1, R1_W2, R2_W1, R2_b1, R2_W2):
    B, E = x.shape
    grid = (E // _BLK,)
    full = lambda shape: pl.BlockSpec(shape, lambda i: tuple(0 for _ in shape))
    return pl.pallas_call(
        _edge_stage_body,
        grid=grid,
        in_specs=[
            pl.BlockSpec((B, _BLK), lambda i: (0, i)),
            pl.BlockSpec((B, _BLK, 3), lambda i: (0, i, 0)),
            full((8, 64)), full((1, 64)), full((64, 64)),
            full((8, 64)), full((1, 64)), full((64, 64)),
        ],
        out_specs=[
            pl.BlockSpec((B, _BLK, 16), lambda i: (0, i, 0)),
            pl.BlockSpec((B, _BLK, 64), lambda i: (0, i, 0)),
            pl.BlockSpec((B, _BLK, 64), lambda i: (0, i, 0)),
        ],
        out_shape=[
            jax.ShapeDtypeStruct((B, E, 16), jnp.float32),
            jax.ShapeDtypeStruct((B, E, 64), jnp.float32),
            jax.ShapeDtypeStruct((B, E, 64), jnp.float32),
        ],
    )(x, x_v, R1_W1, R1_b1.reshape(1, 64), R1_W2, R2_W1, R2_b1.reshape(1, 64), R2_W2)


def _node_stage_body(na_ref, w_ref, h0_ref):
    na = na_ref[...]
    w = w_ref[...]
    for b in range(na.shape[0]):
        h0_ref[b] = jnp.dot(na[b], w, preferred_element_type=jnp.float32)


def _node_stage(node_attr, W_node):
    B, N, Z = node_attr.shape
    NBLK = 2000
    return pl.pallas_call(
        _node_stage_body,
        grid=(N // NBLK,),
        in_specs=[
            pl.BlockSpec((B, NBLK, Z), lambda i: (0, i, 0)),
            pl.BlockSpec((Z, 64), lambda i: (0, 0)),
        ],
        out_specs=pl.BlockSpec((B, NBLK, 64), lambda i: (0, i, 0)),
        out_shape=jax.ShapeDtypeStruct((B, N, 64), jnp.float32),
    )(node_attr, W_node)


def kernel(x, x_v, node_attr, edge_index, W_node, R1_W1, R1_b1, R1_W2,
           r1_read, U1, R2_W1, R2_b1, R2_W2, r2_read):
    B, E = x.shape
    N = node_attr.shape[1]
    C = W_node.shape[1]

    Y16, R1, R2 = _edge_stage(x, x_v, R1_W1, R1_b1, R1_W2, R2_W1, R2_b1, R2_W2)
    h0 = _node_stage(node_attr, W_node)

    src = edge_index[:, 0, :]
    dst = edge_index[:, 1, :]

    def per_graph(h0b, Yb16, R1b, R2b, srcb, dstb):
        Yb = Yb16[:, :9]
        a = h0b[srcb] * R1b
        out1 = jnp.einsum('em,e->m', Yb, a @ r1_read)
        eph = (a[:, :, None] * Yb[:, None, :]).reshape(E, C * 9)
        A = jax.ops.segment_sum(eph, dstb, num_segments=N)
        G = A[srcb].reshape(E, C, 9)
        p = jnp.einsum('edm,em->ed', G, Yb)
        b2 = (p @ U1) * R2b
        out2 = jnp.einsum('em,e->m', Yb, b2 @ r2_read)
        return out1 + out2

    return jax.vmap(per_graph)(h0, Y16, R1, R2, src, dst)
